# Initial kernel scaffold; baseline (speedup 1.0000x reference)
import functools

import jax
import jax.numpy as jnp
from jax import lax
from jax.experimental import pallas as pl
from jax.experimental.pallas import tpu as pltpu
from jax.experimental.pallas import tpu_sc as plsc

N = 10000
E = 320000
IN_DIM = 128
OUT_DIM = 16

NC = 2
NS = 16
NW = NC * NS
NPAD = 10240
ET = E // NW
ETP = 10240
ROWS = ETP // 128
NV = ET // 16
NODES_PER_TILE = NPAD // NS
BS = 1024


def _zsd_body(th_ref, h_ref, w_ref, a_ref, z_ref, s_ref, d_ref):
    th = th_ref[0, 0]
    w = w_ref[...]
    wm = w * (jnp.abs(w) > th).astype(w.dtype)
    z = lax.dot_general(h_ref[...], wm, (((1,), (1,)), ((), ())),
                        preferred_element_type=jnp.float32)
    a = a_ref[...]
    am = a * (jnp.abs(a) > th).astype(a.dtype)
    a1 = am[:, :OUT_DIM]
    a2 = am[:, OUT_DIM:]
    z_ref[...] = z
    s_ref[...] = jnp.sum(z * a1, axis=1, keepdims=True)
    d_ref[...] = jnp.sum(z * a2, axis=1, keepdims=True)


_zsd_call = pl.pallas_call(
    _zsd_body,
    grid=(NPAD // BS,),
    in_specs=[
        pl.BlockSpec((1, 1), lambda i: (0, 0)),
        pl.BlockSpec((BS, IN_DIM), lambda i: (i, 0)),
        pl.BlockSpec((OUT_DIM, IN_DIM), lambda i: (0, 0)),
        pl.BlockSpec((1, 2 * OUT_DIM), lambda i: (0, 0)),
    ],
    out_specs=[
        pl.BlockSpec((BS, OUT_DIM), lambda i: (i, 0)),
        pl.BlockSpec((BS, 1), lambda i: (i, 0)),
        pl.BlockSpec((BS, 1), lambda i: (i, 0)),
    ],
    out_shape=[
        jax.ShapeDtypeStruct((NPAD, OUT_DIM), jnp.float32),
        jax.ShapeDtypeStruct((NPAD, 1), jnp.float32),
        jax.ShapeDtypeStruct((NPAD, 1), jnp.float32),
    ],
)


def _edge_body(srcp, dstp, z_hbm, s_hbm, d_hbm, accp, denp,
               s_loc, d_loc, src_loc, dst_loc, ex_loc, den_loc,
               zrow, wrow, zbuf, acc_sh, sem):
    cid = lax.axis_index("c")
    sid = lax.axis_index("s")
    wid = cid * NS + sid
    zero16 = jnp.zeros((16,), jnp.float32)

    def zloop(i, carry):
        zbuf[i, :] = zero16
        den_loc[pl.ds(i * 16, 16)] = zero16
        ex_loc[i // 8, pl.ds((i % 8) * 16, 16)] = zero16
        return carry
    lax.fori_loop(0, NPAD // 16, zloop, 0)

    pltpu.sync_copy(s_hbm, s_loc)
    pltpu.sync_copy(d_hbm, d_loc)
    pltpu.sync_copy(srcp.at[wid], src_loc)
    pltpu.sync_copy(dstp.at[wid], dst_loc)
    pltpu.sync_copy(zbuf, acc_sh.at[pl.ds(sid * NODES_PER_TILE, NODES_PER_TILE)])
    plsc.subcore_barrier()

    def body1(i, carry):
        j = i // 8
        off = (i % 8) * 16
        si = src_loc[j, pl.ds(off, 16)]
        di = dst_loc[j, pl.ds(off, 16)]
        sv = plsc.load_gather(s_loc, [si])
        dv = plsc.load_gather(d_loc, [di])
        e = sv + dv
        e = jnp.where(e > 0, e, e * jnp.float32(0.01))
        ex = jnp.exp(e)
        ex_loc[j, pl.ds(off, 16)] = ex
        plsc.addupdate_scatter(den_loc, [di], ex)
        return carry
    lax.fori_loop(0, NV, body1, 0)
    pltpu.sync_copy(den_loc, denp.at[cid, sid])

    def body2(j, carry):
        pltpu.async_copy(z_hbm.at[src_loc.at[j]], zrow, sem).wait()
        def scale(r, c2):
            wrow[r, :] = zrow[r, :] * ex_loc[j, r]
            return c2
        lax.fori_loop(0, 128, scale, 0)
        pltpu.sync_copy(wrow, acc_sh.at[dst_loc.at[j]], add=True)
        return carry
    lax.fori_loop(0, ROWS, body2, 0)
    plsc.subcore_barrier()

    nbase = sid * NODES_PER_TILE
    pltpu.sync_copy(acc_sh.at[pl.ds(nbase, NODES_PER_TILE)],
                    accp.at[cid, pl.ds(nbase, NODES_PER_TILE)])


_edge_call = functools.partial(
    pl.kernel,
    out_type=(jax.ShapeDtypeStruct((NC, NPAD, OUT_DIM), jnp.float32),
              jax.ShapeDtypeStruct((NC, NS, NPAD), jnp.float32)),
    mesh=plsc.VectorSubcoreMesh(core_axis_name="c", subcore_axis_name="s"),
    scratch_types=[
        pltpu.VMEM((NPAD,), jnp.float32),
        pltpu.VMEM((NPAD,), jnp.float32),
        pltpu.VMEM((ROWS, 128), jnp.int32),
        pltpu.VMEM((ROWS, 128), jnp.int32),
        pltpu.VMEM((ROWS, 128), jnp.float32),
        pltpu.VMEM((NPAD,), jnp.float32),
        pltpu.VMEM((128, OUT_DIM), jnp.float32),
        pltpu.VMEM((128, OUT_DIM), jnp.float32),
        pltpu.VMEM((NODES_PER_TILE, OUT_DIM), jnp.float32),
        pltpu.VMEM_SHARED((NPAD, OUT_DIM), jnp.float32),
        pltpu.SemaphoreType.DMA,
    ],
)(_edge_body)


def _combine_body(accp_ref, denp_ref, out_ref):
    a = accp_ref[0] + accp_ref[1]
    den = jnp.sum(denp_ref[...], axis=(0, 1))
    y = a / (den[:, None] + jnp.float32(1e-16))
    out_ref[...] = jnp.where(y > 0, y, jnp.expm1(y))


_combine_call = pl.pallas_call(
    _combine_body,
    out_shape=jax.ShapeDtypeStruct((NPAD, OUT_DIM), jnp.float32),
)


def kernel(h, edge_index, threshold, W, A):
    src = edge_index[0]
    dst = edge_index[1]
    hp = jnp.pad(h, ((0, NPAD - N), (0, 0)))
    th2 = jnp.reshape(threshold.astype(jnp.float32), (1, 1))
    z, s2, d2 = _zsd_call(th2, hp, W, A)
    s = s2.reshape(NPAD)
    d = d2.reshape(NPAD)
    srcp = jnp.pad(src.reshape(NW, ET), ((0, 0), (0, ETP - ET)))
    dstp = jnp.pad(dst.reshape(NW, ET), ((0, 0), (0, ETP - ET)))
    srcp = srcp.reshape(NW, ROWS, 128)
    dstp = dstp.reshape(NW, ROWS, 128)
    accp, denp = _edge_call(srcp, dstp, z, s, d)
    out = _combine_call(accp, denp)
    return out[:N]


# speedup vs baseline: 31.3258x; 31.3258x over previous
import functools

import jax
import jax.numpy as jnp
from jax import lax
from jax.experimental import pallas as pl
from jax.experimental.pallas import tpu as pltpu
from jax.experimental.pallas import tpu_sc as plsc

N = 10000
E = 320000
IN_DIM = 128
OUT_DIM = 16

NC = 2
NS = 16
NW = NC * NS
NPAD = 10240
ET = E // NW
ETP = 10240
ROWS = ETP // 128
NV = ET // 16
NODES_PER_TILE = NPAD // NS
BS = 1024


def _zsd_body(th_ref, h_ref, w_ref, a_ref, z_ref, s_ref, d_ref):
    th = th_ref[0, 0]
    w = w_ref[...]
    wm = w * (jnp.abs(w) > th).astype(w.dtype)
    z = lax.dot_general(h_ref[...], wm, (((1,), (1,)), ((), ())),
                        preferred_element_type=jnp.float32)
    a = a_ref[...]
    am = a * (jnp.abs(a) > th).astype(a.dtype)
    a1 = am[:, :OUT_DIM]
    a2 = am[:, OUT_DIM:]
    z_ref[...] = z
    s_ref[...] = jnp.sum(z * a1, axis=1, keepdims=True)
    d_ref[...] = jnp.sum(z * a2, axis=1, keepdims=True)


_zsd_call = pl.pallas_call(
    _zsd_body,
    grid=(NPAD // BS,),
    in_specs=[
        pl.BlockSpec((1, 1), lambda i: (0, 0)),
        pl.BlockSpec((BS, IN_DIM), lambda i: (i, 0)),
        pl.BlockSpec((OUT_DIM, IN_DIM), lambda i: (0, 0)),
        pl.BlockSpec((1, 2 * OUT_DIM), lambda i: (0, 0)),
    ],
    out_specs=[
        pl.BlockSpec((BS, OUT_DIM), lambda i: (i, 0)),
        pl.BlockSpec((BS, 1), lambda i: (i, 0)),
        pl.BlockSpec((BS, 1), lambda i: (i, 0)),
    ],
    out_shape=[
        jax.ShapeDtypeStruct((NPAD, OUT_DIM), jnp.float32),
        jax.ShapeDtypeStruct((NPAD, 1), jnp.float32),
        jax.ShapeDtypeStruct((NPAD, 1), jnp.float32),
    ],
)


def _edge_body(srcp, dstp, z_hbm, s_hbm, d_hbm, accp, denp,
               s_loc, d_loc, src_loc, dst_loc, ex_loc, den_loc,
               zrow, wrow, zbuf, acc_sh, sem):
    cid = lax.axis_index("c")
    sid = lax.axis_index("s")
    wid = cid * NS + sid
    zero16 = jnp.zeros((16,), jnp.float32)

    def zloop(i, carry):
        zbuf[i, :] = zero16
        den_loc[pl.ds(i * 16, 16)] = zero16
        ex_loc[i // 8, pl.ds((i % 8) * 16, 16)] = zero16
        return carry
    lax.fori_loop(0, NPAD // 16, zloop, 0)

    pltpu.sync_copy(s_hbm, s_loc)
    pltpu.sync_copy(d_hbm, d_loc)
    pltpu.sync_copy(srcp.at[wid], src_loc)
    pltpu.sync_copy(dstp.at[wid], dst_loc)
    pltpu.sync_copy(zbuf, acc_sh.at[pl.ds(sid * NODES_PER_TILE, NODES_PER_TILE)])
    plsc.subcore_barrier()

    def body1(i, carry):
        j = i // 8
        off = (i % 8) * 16
        si = src_loc[j, pl.ds(off, 16)]
        di = dst_loc[j, pl.ds(off, 16)]
        sv = plsc.load_gather(s_loc, [si])
        dv = plsc.load_gather(d_loc, [di])
        e = sv + dv
        e = jnp.where(e > 0, e, e * jnp.float32(0.01))
        ex = jnp.exp(e)
        ex_loc[j, pl.ds(off, 16)] = ex
        plsc.addupdate_scatter(den_loc, [di], ex)
        return carry
    lax.fori_loop(0, NV, body1, 0)
    pltpu.sync_copy(den_loc, denp.at[cid, sid])

    def body2(j, carry):
        pltpu.async_copy(z_hbm.at[src_loc.at[j]], zrow, sem).wait()
        def scale(k, c2):
            exv = ex_loc[j, pl.ds(k * 16, 16)]
            base = k * 16
            for t in range(16):
                wrow[base + t, :] = zrow[base + t, :] * exv[t]
            return c2
        lax.fori_loop(0, 8, scale, 0)
        pltpu.sync_copy(wrow, acc_sh.at[dst_loc.at[j]], add=True)
        return carry
    lax.fori_loop(0, ROWS, body2, 0)
    plsc.subcore_barrier()

    nbase = sid * NODES_PER_TILE
    pltpu.sync_copy(acc_sh.at[pl.ds(nbase, NODES_PER_TILE)],
                    accp.at[cid, pl.ds(nbase, NODES_PER_TILE)])


_edge_call = functools.partial(
    pl.kernel,
    out_type=(jax.ShapeDtypeStruct((NC, NPAD, OUT_DIM), jnp.float32),
              jax.ShapeDtypeStruct((NC, NS, NPAD), jnp.float32)),
    mesh=plsc.VectorSubcoreMesh(core_axis_name="c", subcore_axis_name="s",
                                num_cores=NC, num_subcores=NS),
    scratch_types=[
        pltpu.VMEM((NPAD,), jnp.float32),
        pltpu.VMEM((NPAD,), jnp.float32),
        pltpu.VMEM((ROWS, 128), jnp.int32),
        pltpu.VMEM((ROWS, 128), jnp.int32),
        pltpu.VMEM((ROWS, 128), jnp.float32),
        pltpu.VMEM((NPAD,), jnp.float32),
        pltpu.VMEM((128, OUT_DIM), jnp.float32),
        pltpu.VMEM((128, OUT_DIM), jnp.float32),
        pltpu.VMEM((NODES_PER_TILE, OUT_DIM), jnp.float32),
        pltpu.VMEM_SHARED((NPAD, OUT_DIM), jnp.float32),
        pltpu.SemaphoreType.DMA,
    ],
    compiler_params=pltpu.CompilerParams(needs_layout_passes=False,
                                         use_tc_tiling_on_sc=False),
)(_edge_body)


def _combine_body(accp_ref, denp_ref, out_ref):
    a = accp_ref[0] + accp_ref[1]
    den = jnp.sum(denp_ref[...], axis=(0, 1))
    y = a / (den[:, None] + jnp.float32(1e-16))
    out_ref[...] = jnp.where(y > 0, y, jnp.exp(y) - jnp.float32(1.0))


_combine_call = pl.pallas_call(
    _combine_body,
    out_shape=jax.ShapeDtypeStruct((NPAD, OUT_DIM), jnp.float32),
)


def kernel(h, edge_index, threshold, W, A):
    src = edge_index[0]
    dst = edge_index[1]
    hp = jnp.pad(h, ((0, NPAD - N), (0, 0)))
    th2 = jnp.reshape(threshold.astype(jnp.float32), (1, 1))
    z, s2, d2 = _zsd_call(th2, hp, W, A)
    s = s2.reshape(NPAD)
    d = d2.reshape(NPAD)
    srcp = jnp.pad(src.reshape(NW, ET), ((0, 0), (0, ETP - ET)))
    dstp = jnp.pad(dst.reshape(NW, ET), ((0, 0), (0, ETP - ET)))
    srcp = srcp.reshape(NW, ROWS, 128)
    dstp = dstp.reshape(NW, ROWS, 128)
    accp, denp = _edge_call(srcp, dstp, z, s, d)
    out = _combine_call(accp, denp)
    return out[:N]


# speedup vs baseline: 39.4214x; 1.2584x over previous
import functools

import jax
import jax.numpy as jnp
from jax import lax
from jax.experimental import pallas as pl
from jax.experimental.pallas import tpu as pltpu
from jax.experimental.pallas import tpu_sc as plsc

N = 10000
E = 320000
IN_DIM = 128
OUT_DIM = 16

NC = 2
NS = 16
NW = NC * NS
NPAD = 10240
ET = E // NW
ETP = 10240
ROWS = ETP // 128
NV = ET // 16
NODES_PER_TILE = NPAD // NS
BS = 1024


def _zsd_body(th_ref, h_ref, w_ref, a_ref, z_ref, s_ref, d_ref):
    th = th_ref[0, 0]
    w = w_ref[...]
    wm = w * (jnp.abs(w) > th).astype(w.dtype)
    z = lax.dot_general(h_ref[...], wm, (((1,), (1,)), ((), ())),
                        preferred_element_type=jnp.float32)
    a = a_ref[...]
    am = a * (jnp.abs(a) > th).astype(a.dtype)
    a1 = am[:, :OUT_DIM]
    a2 = am[:, OUT_DIM:]
    z_ref[...] = z
    s_ref[...] = jnp.sum(z * a1, axis=1, keepdims=True)
    d_ref[...] = jnp.sum(z * a2, axis=1, keepdims=True)


_zsd_call = pl.pallas_call(
    _zsd_body,
    grid=(NPAD // BS,),
    in_specs=[
        pl.BlockSpec((1, 1), lambda i: (0, 0)),
        pl.BlockSpec((BS, IN_DIM), lambda i: (i, 0)),
        pl.BlockSpec((OUT_DIM, IN_DIM), lambda i: (0, 0)),
        pl.BlockSpec((1, 2 * OUT_DIM), lambda i: (0, 0)),
    ],
    out_specs=[
        pl.BlockSpec((BS, OUT_DIM), lambda i: (i, 0)),
        pl.BlockSpec((BS, 1), lambda i: (i, 0)),
        pl.BlockSpec((BS, 1), lambda i: (i, 0)),
    ],
    out_shape=[
        jax.ShapeDtypeStruct((NPAD, OUT_DIM), jnp.float32),
        jax.ShapeDtypeStruct((NPAD, 1), jnp.float32),
        jax.ShapeDtypeStruct((NPAD, 1), jnp.float32),
    ],
)


def _edge_body(srcp, dstp, z_hbm, s_hbm, d_hbm, accp, denp,
               s_loc, d_loc, src_loc, dst_loc, ex_loc, den_loc,
               zrow0, zrow1, wrow0, wrow1, zbuf, acc_sh,
               gsem0, gsem1, ssem0, ssem1, sem):
    cid = lax.axis_index("c")
    sid = lax.axis_index("s")
    wid = cid * NS + sid
    zero16 = jnp.zeros((16,), jnp.float32)

    cp_s = pltpu.async_copy(s_hbm, s_loc, gsem0)
    cp_d = pltpu.async_copy(d_hbm, d_loc, gsem1)
    cp_src = pltpu.async_copy(srcp.at[wid], src_loc, ssem0)
    cp_dst = pltpu.async_copy(dstp.at[wid], dst_loc, ssem1)

    def zloop(i, carry):
        zbuf[i, :] = zero16
        den_loc[pl.ds(i * 16, 16)] = zero16
        ex_loc[i // 8, pl.ds((i % 8) * 16, 16)] = zero16
        return carry
    lax.fori_loop(0, NPAD // 16, zloop, 0)

    pltpu.sync_copy(zbuf, acc_sh.at[pl.ds(sid * NODES_PER_TILE, NODES_PER_TILE)])
    cp_s.wait()
    cp_d.wait()
    cp_src.wait()
    cp_dst.wait()
    plsc.subcore_barrier()

    def body1(i, carry):
        j = i // 8
        off = (i % 8) * 16
        si = src_loc[j, pl.ds(off, 16)]
        di = dst_loc[j, pl.ds(off, 16)]
        sv = plsc.load_gather(s_loc, [si])
        dv = plsc.load_gather(d_loc, [di])
        e = sv + dv
        e = jnp.where(e > 0, e, e * jnp.float32(0.01))
        ex = jnp.exp(e)
        ex_loc[j, pl.ds(off, 16)] = ex
        plsc.addupdate_scatter(den_loc, [di], ex)
        return carry
    lax.fori_loop(0, NV, body1, 0)
    pltpu.sync_copy(den_loc, denp.at[cid, sid])

    bufs = ((zrow0, wrow0, gsem0, ssem0), (zrow1, wrow1, gsem1, ssem1))
    for b, (zb, wb, gs, ss) in enumerate(bufs):
        pltpu.async_copy(z_hbm.at[src_loc.at[b]], zb, gs)

    def body2(jj, carry):
        for b, (zb, wb, gs, ss) in enumerate(bufs):
            j = jj * 2 + b
            pltpu.make_async_copy(z_hbm.at[src_loc.at[j]], zb, gs).wait()

            @pl.when(jj > 0)
            def _():
                pltpu.make_async_copy(wb, acc_sh.at[dst_loc.at[j]], ss).wait()

            def scale(k, c2):
                exv = ex_loc[j, pl.ds(k * 16, 16)]
                base = k * 16
                for t in range(16):
                    wb[base + t, :] = zb[base + t, :] * exv[t]
                return c2
            lax.fori_loop(0, 8, scale, 0)
            pltpu.async_copy(wb, acc_sh.at[dst_loc.at[j]], ss, add=True)

            @pl.when(j + 2 < ROWS)
            def _():
                pltpu.async_copy(z_hbm.at[src_loc.at[j + 2]], zb, gs)
        return carry
    lax.fori_loop(0, ROWS // 2, body2, 0)
    for b, (zb, wb, gs, ss) in enumerate(bufs):
        pltpu.make_async_copy(wb, acc_sh.at[dst_loc.at[b]], ss).wait()
    plsc.subcore_barrier()

    nbase = sid * NODES_PER_TILE
    pltpu.sync_copy(acc_sh.at[pl.ds(nbase, NODES_PER_TILE)],
                    accp.at[cid, pl.ds(nbase, NODES_PER_TILE)])


_edge_call = functools.partial(
    pl.kernel,
    out_type=(jax.ShapeDtypeStruct((NC, NPAD, OUT_DIM), jnp.float32),
              jax.ShapeDtypeStruct((NC, NS, NPAD), jnp.float32)),
    mesh=plsc.VectorSubcoreMesh(core_axis_name="c", subcore_axis_name="s",
                                num_cores=NC, num_subcores=NS),
    scratch_types=[
        pltpu.VMEM((NPAD,), jnp.float32),
        pltpu.VMEM((NPAD,), jnp.float32),
        pltpu.VMEM((ROWS, 128), jnp.int32),
        pltpu.VMEM((ROWS, 128), jnp.int32),
        pltpu.VMEM((ROWS, 128), jnp.float32),
        pltpu.VMEM((NPAD,), jnp.float32),
        pltpu.VMEM((128, OUT_DIM), jnp.float32),
        pltpu.VMEM((128, OUT_DIM), jnp.float32),
        pltpu.VMEM((128, OUT_DIM), jnp.float32),
        pltpu.VMEM((128, OUT_DIM), jnp.float32),
        pltpu.VMEM((NODES_PER_TILE, OUT_DIM), jnp.float32),
        pltpu.VMEM_SHARED((NPAD, OUT_DIM), jnp.float32),
        pltpu.SemaphoreType.DMA,
        pltpu.SemaphoreType.DMA,
        pltpu.SemaphoreType.DMA,
        pltpu.SemaphoreType.DMA,
        pltpu.SemaphoreType.DMA,
    ],
    compiler_params=pltpu.CompilerParams(needs_layout_passes=False,
                                         use_tc_tiling_on_sc=False),
)(_edge_body)


def _combine_body(accp_ref, denp_ref, out_ref):
    a = accp_ref[0] + accp_ref[1]
    den = jnp.sum(denp_ref[...], axis=(0, 1))
    y = a / (den[:, None] + jnp.float32(1e-16))
    out_ref[...] = jnp.where(y > 0, y, jnp.exp(y) - jnp.float32(1.0))


_combine_call = pl.pallas_call(
    _combine_body,
    out_shape=jax.ShapeDtypeStruct((NPAD, OUT_DIM), jnp.float32),
)


def kernel(h, edge_index, threshold, W, A):
    src = edge_index[0]
    dst = edge_index[1]
    hp = jnp.pad(h, ((0, NPAD - N), (0, 0)))
    th2 = jnp.reshape(threshold.astype(jnp.float32), (1, 1))
    z, s2, d2 = _zsd_call(th2, hp, W, A)
    s = s2.reshape(NPAD)
    d = d2.reshape(NPAD)
    srcp = jnp.pad(src.reshape(NW, ET), ((0, 0), (0, ETP - ET)))
    dstp = jnp.pad(dst.reshape(NW, ET), ((0, 0), (0, ETP - ET)))
    srcp = srcp.reshape(NW, ROWS, 128)
    dstp = dstp.reshape(NW, ROWS, 128)
    accp, denp = _edge_call(srcp, dstp, z, s, d)
    out = _combine_call(accp, denp)
    return out[:N]
